# Initial kernel scaffold; baseline (speedup 1.0000x reference)
import jax
import jax.numpy as jnp
from jax.experimental import pallas as pl


def kernel(hidden_states, gate_w, wg, wu, wd):
    raise NotImplementedError("write your pallas kernel here")


# speedup vs baseline: 1.1881x; 1.1881x over previous
import jax
import jax.numpy as jnp
from jax.experimental import pallas as pl
from jax.experimental.pallas import tpu as pltpu

NUM_EXPERTS = 8
HIDDEN = 1024
FFN = 2048
T = 2048
T_BLK = 256
NT = T // T_BLK
F_BLK = 1024
NF = FFN // F_BLK


def _router_weights(h_blk, gate_w):
    logits = jax.lax.dot_general(
        h_blk, gate_w, (((1,), (1,)), ((), ())),
        preferred_element_type=jnp.float32)
    m = jnp.max(logits, axis=-1, keepdims=True)
    ex = jnp.exp(logits - m)
    p = ex / jnp.sum(ex, axis=-1, keepdims=True)
    idx = jax.lax.broadcasted_iota(jnp.int32, p.shape, 1)
    BIG = jnp.int32(NUM_EXPERTS)
    m1 = jnp.max(p, axis=-1, keepdims=True)
    i1 = jnp.min(jnp.where(p == m1, idx, BIG), axis=-1, keepdims=True)
    mask1 = idx == i1
    p2 = jnp.where(mask1, -1.0, p)
    m2 = jnp.max(p2, axis=-1, keepdims=True)
    i2 = jnp.min(jnp.where(p2 == m2, idx, BIG), axis=-1, keepdims=True)
    mask2 = idx == i2
    denom = m1 + m2
    w = (jnp.where(mask1, p, 0.0) + jnp.where(mask2, p, 0.0)) / denom
    return w


def _silu(x):
    return x * (1.0 / (1.0 + jnp.exp(-x)))


def _moe_body(h_ref, gate_ref, wg_ref, wu_ref, wd_ref, out_ref, w_ref):
    e = pl.program_id(0)
    f = pl.program_id(1)
    t = pl.program_id(2)
    h = h_ref[pl.ds(t * T_BLK, T_BLK), :]

    @pl.when((e == 0) & (f == 0))
    def _():
        w_ref[pl.ds(t * T_BLK, T_BLK), :] = _router_weights(h, gate_ref[...])

    wg = wg_ref[0]
    wu = wu_ref[0]
    wd = wd_ref[0]
    gate_out = jax.lax.dot_general(h, wg, (((1,), (1,)), ((), ())),
                                   preferred_element_type=jnp.float32)
    up_out = jax.lax.dot_general(h, wu, (((1,), (1,)), ((), ())),
                                 preferred_element_type=jnp.float32)
    act = _silu(gate_out) * up_out
    partial = jax.lax.dot_general(act, wd, (((1,), (1,)), ((), ())),
                                  preferred_element_type=jnp.float32)
    w_blk = w_ref[pl.ds(t * T_BLK, T_BLK), :]
    eidx = jax.lax.broadcasted_iota(jnp.int32, w_blk.shape, 1)
    wcol = jnp.sum(jnp.where(eidx == e, w_blk, 0.0), axis=-1,
                   keepdims=True)
    contrib = partial * wcol

    @pl.when((e == 0) & (f == 0))
    def _():
        out_ref[pl.ds(t * T_BLK, T_BLK), :] = contrib

    @pl.when((e > 0) | (f > 0))
    def _():
        out_ref[pl.ds(t * T_BLK, T_BLK), :] += contrib


@jax.jit
def kernel(hidden_states, gate_w, wg, wu, wd):
    B, S, H = hidden_states.shape
    hidden = hidden_states.reshape(-1, H)
    out = pl.pallas_call(
        _moe_body,
        grid=(NUM_EXPERTS, NF, NT),
        in_specs=[
            pl.BlockSpec((T, HIDDEN), lambda e, f, t: (0, 0)),
            pl.BlockSpec((NUM_EXPERTS, HIDDEN), lambda e, f, t: (0, 0)),
            pl.BlockSpec((1, F_BLK, HIDDEN), lambda e, f, t: (e, f, 0)),
            pl.BlockSpec((1, F_BLK, HIDDEN), lambda e, f, t: (e, f, 0)),
            pl.BlockSpec((1, HIDDEN, F_BLK), lambda e, f, t: (e, 0, f)),
        ],
        out_specs=pl.BlockSpec((T, HIDDEN), lambda e, f, t: (0, 0)),
        out_shape=jax.ShapeDtypeStruct((T, HIDDEN), jnp.float32),
        scratch_shapes=[
            pltpu.VMEM((T, NUM_EXPERTS), jnp.float32),
        ],
        compiler_params=pltpu.CompilerParams(
            dimension_semantics=("arbitrary", "arbitrary", "arbitrary"),
        ),
    )(hidden, gate_w, wg, wu, wd)
    return out.reshape(B, S, H)
